# baseline (speedup 1.0000x reference)
import functools

import jax
import jax.numpy as jnp
from jax import lax
from jax.experimental import pallas as pl
from jax.experimental.pallas import tpu as pltpu
from jax.experimental.pallas import tpu_sc as plsc

N = 10000
E = 320000
D = 128
R = 8
G = 64

NC = 2
NS = 16
NW = NC * NS

EB = 80
EPT_ALL = E // NS
EPT = E // NW
CNT_BINS = N * R
CNT_PAD = 81920
ZROWS = 40
WB_TILES = 10
WB_ROWS = N // WB_TILES

@functools.cache
def _mesh():
    return plsc.VectorSubcoreMesh(
        core_axis_name="c", subcore_axis_name="s", num_cores=NC, num_subcores=NS
    )


def _wid():
    return lax.axis_index("s") * NC + lax.axis_index("c")


EBA = 160
NBLK_A = EPT_ALL // EBA
NBLK_B = EPT // EB


@functools.cache
def _k0():
  return pl.kernel(
    _k0_body,
    out_type=[
        jax.ShapeDtypeStruct((E,), jnp.float32),
        jax.ShapeDtypeStruct((E,), jnp.int32),
    ],
    mesh=_mesh(),
    scratch_types=[
        pltpu.VMEM_SHARED((CNT_PAD,), jnp.float32),
        pltpu.VMEM((3 * EB,), jnp.float32),
        pltpu.VMEM((5120,), jnp.float32),
        pltpu.VMEM((2 * EBA,), jnp.int32),
        pltpu.VMEM((2 * EBA,), jnp.int32),
        pltpu.VMEM((2 * EB,), jnp.int32),
        pltpu.VMEM((4, EB), jnp.int32),
        pltpu.VMEM((3 * EB,), jnp.int32),
        pltpu.VMEM((EB,), jnp.float32),
        pltpu.VMEM((3 * EB,), jnp.float32),
        pltpu.VMEM((3 * EB,), jnp.int32),
        pltpu.SemaphoreType.DMA,
        pltpu.SemaphoreType.DMA,
        pltpu.SemaphoreType.DMA,
        pltpu.SemaphoreType.DMA,
        pltpu.SemaphoreType.DMA,
        pltpu.SemaphoreType.DMA,
        pltpu.SemaphoreType.DMA,
    ],
  )


def _k0_body(src_hbm, t_hbm, dst_hbm, scale_hbm, ridx_hbm,
             cnt_sh, cntv, zbuf, dstv, tv, srcv, bidxA, bidxB, ones,
             sbuf, rbuf, semI0, semI1, semG0, semG1, semG2, semS0, semS1):
    sid = lax.axis_index("s")
    semG = (semG0, semG1, semG2)

    def _off(x):
        return x if isinstance(x, int) else pl.multiple_of(x, 8)

    def _sel(p, q, fn):
        if isinstance(p, int):
            if p == q:
                fn()
        else:
            pl.when(p == q)(fn)

    def _zfill(j, _):
        zbuf[pl.ds(j * 16, 16)] = jnp.zeros((16,), jnp.float32)
        return 0

    lax.fori_loop(0, 5120 // 16, _zfill, 0)
    pltpu.sync_copy(zbuf, cnt_sh.at[pl.ds(sid * 5120, 5120)])
    for j in range(EB // 16):
        ones[pl.ds(j * 16, 16)] = jnp.ones((16,), jnp.float32)
    plsc.subcore_barrier()

    base_a = sid * EPT_ALL

    def fsA(s):
        p, b, po = s % 2, _off(base_a + s * EBA), _off((s % 2) * EBA)
        for semIx, q in ((semI0, 0), (semI1, 1)):
            def _f(semIx=semIx):
                pltpu.async_copy(dst_hbm.at[pl.ds(b, EBA)],
                                 dstv.at[pl.ds(po, EBA)], semIx)
                pltpu.async_copy(t_hbm.at[pl.ds(b, EBA)],
                                 tv.at[pl.ds(po, EBA)], semIx)
            _sel(p, q, _f)

    def wsA(s):
        p, po = s % 2, _off((s % 2) * EBA)
        for semIx, q in ((semI0, 0), (semI1, 1)):
            def _f(semIx=semIx):
                pltpu.make_async_copy(dst_hbm.at[pl.ds(0, EBA)],
                                      dstv.at[pl.ds(po, EBA)], semIx).wait()
                pltpu.make_async_copy(t_hbm.at[pl.ds(0, EBA)],
                                      tv.at[pl.ds(po, EBA)], semIx).wait()
            _sel(p, q, _f)

    def fscatA(s):
        p = s % 2
        for semSx, q in ((semS0, 0), (semS1, 1)):
            def _f(semSx=semSx):
                pltpu.async_copy(ones, cnt_sh.at[bidxA.at[2 * p]],
                                 semSx, add=True)
                pltpu.async_copy(ones, cnt_sh.at[bidxA.at[2 * p + 1]],
                                 semSx, add=True)
            _sel(p, q, _f)

    def wscatA(s):
        p = s % 2
        for semSx, q in ((semS0, 0), (semS1, 1)):
            def _f(semSx=semSx):
                pltpu.make_async_copy(ones, cnt_sh.at[bidxA.at[2 * p]],
                                      semSx).wait()
                pltpu.make_async_copy(ones, cnt_sh.at[bidxA.at[2 * p + 1]],
                                      semSx).wait()
            _sel(p, q, _f)

    fsA(0)

    def _iterA(s, _):
        p = s % 2
        po = _off(p * EBA)
        wsA(s)

        @pl.when(s >= 2)
        def _():
            wscatA(s - 2)

        for j in range(EBA // 16):
            sl = pl.ds(po + j * 16, 16)
            bidxA[2 * p + j // 5, pl.ds((j % 5) * 16, 16)] = (
                dstv[sl] * R + tv[sl])
        fscatA(s)

        @pl.when(s < NBLK_A - 1)
        def _():
            fsA(s + 1)

        return 0

    lax.fori_loop(0, NBLK_A, _iterA, 0)
    wscatA(NBLK_A - 2)
    wscatA(NBLK_A - 1)
    plsc.subcore_barrier()

    base_c = _wid() * EPT

    def fsB(s):
        p, b, po = s % 2, _off(base_c + s * EB), _off((s % 2) * EB)
        for semI, q in ((semI0, 0), (semI1, 1)):
            def _f(semI=semI):
                pltpu.async_copy(src_hbm.at[pl.ds(b, EB)],
                                 srcv.at[pl.ds(po, EB)], semI)
                pltpu.async_copy(t_hbm.at[pl.ds(b, EB)],
                                 tv.at[pl.ds(po, EB)], semI)
                pltpu.async_copy(dst_hbm.at[pl.ds(b, EB)],
                                 dstv.at[pl.ds(po, EB)], semI)
            _sel(p, q, _f)

    def wsB(s):
        p, po = s % 2, _off((s % 2) * EB)
        for semI, q in ((semI0, 0), (semI1, 1)):
            def _f(semI=semI):
                pltpu.make_async_copy(src_hbm.at[pl.ds(0, EB)],
                                      srcv.at[pl.ds(po, EB)], semI).wait()
                pltpu.make_async_copy(t_hbm.at[pl.ds(0, EB)],
                                      tv.at[pl.ds(po, EB)], semI).wait()
                pltpu.make_async_copy(dst_hbm.at[pl.ds(0, EB)],
                                      dstv.at[pl.ds(po, EB)], semI).wait()
            _sel(p, q, _f)

    def _sel3(v, fn):
        if isinstance(v, int):
            fn(v % 3)
        else:
            m = v % 3
            for i in range(3):
                pl.when(m == i)(functools.partial(fn, i))

    def fgB(s):
        go = _off((s % 3) * EB)

        def _f(i):
            pltpu.async_copy(cnt_sh.at[bidxB.at[pl.ds(go, EB)]],
                             cntv.at[pl.ds(go, EB)], semG[i])
        _sel3(s, _f)

    def wgB(s):
        go = _off((s % 3) * EB)

        def _f(i):
            pltpu.make_async_copy(cnt_sh.at[bidxB.at[pl.ds(go, EB)]],
                                  cntv.at[pl.ds(go, EB)], semG[i]).wait()
        _sel3(s, _f)

    def fwB(s):
        b, go = _off(base_c + s * EB), _off((s % 3) * EB)
        for semSx, q in ((semS0, 0), (semS1, 1)):
            def _f(semSx=semSx):
                pltpu.async_copy(sbuf.at[pl.ds(go, EB)],
                                 scale_hbm.at[pl.ds(b, EB)], semSx)
                pltpu.async_copy(rbuf.at[pl.ds(go, EB)],
                                 ridx_hbm.at[pl.ds(b, EB)], semSx)
            _sel(s % 2, q, _f)

    def wwB(s):
        go = _off((s % 3) * EB)
        for semSx, q in ((semS0, 0), (semS1, 1)):
            def _f(semSx=semSx):
                pltpu.make_async_copy(sbuf.at[pl.ds(go, EB)],
                                      scale_hbm.at[pl.ds(0, EB)],
                                      semSx).wait()
                pltpu.make_async_copy(rbuf.at[pl.ds(go, EB)],
                                      ridx_hbm.at[pl.ds(0, EB)],
                                      semSx).wait()
            _sel(s % 2, q, _f)

    def emitB(s):
        go = _off((s % 3) * EB)
        for j in range(EB // 16):
            sl = pl.ds(go + j * 16, 16)
            sbuf[sl] = 1.0 / jnp.maximum(cntv[sl], 1.0)

    fsB(0)

    def _iterB(s, _):
        p = s % 2
        po = _off(p * EB)
        go = _off((s % 3) * EB)
        wsB(s)

        @pl.when(s >= 3)
        def _():
            wwB(s - 3)

        for j in range(EB // 16):
            sl = pl.ds(po + j * 16, 16)
            gl = pl.ds(go + j * 16, 16)
            t16 = tv[sl]
            rbuf[gl] = t16 * N + srcv[sl]
            bidxB[gl] = dstv[sl] * R + t16
        fgB(s)

        @pl.when(s >= 2)
        def _():
            wgB(s - 2)
            emitB(s - 2)
            fwB(s - 2)

        @pl.when(s < NBLK_B - 1)
        def _():
            fsB(s + 1)

        return 0

    lax.fori_loop(0, NBLK_B, _iterB, 0)
    for s in (NBLK_B - 2, NBLK_B - 1):
        wgB(s)
        emitB(s)
        fwB(s)
    for s in (NBLK_B - 3, NBLK_B - 2, NBLK_B - 1):
        wwB(s)


NBLK_E = EPT // EB


RROWS = 4
RSTG = 5


@functools.cache
def _kB():
  return pl.kernel(
    _kB_body,
    out_type=jax.ShapeDtypeStruct((NC, N, D), jnp.float32),
    mesh=_mesh(),
    scratch_types=[
        pltpu.VMEM_SHARED((N, D), jnp.float32),
        pltpu.VMEM((RROWS * EB, D), jnp.float32),
        pltpu.VMEM((RSTG * EB,), jnp.int32),
        pltpu.VMEM((RSTG * EB,), jnp.float32),
        pltpu.VMEM((RSTG, EB), jnp.int32),
        pltpu.SemaphoreType.DMA,
        pltpu.SemaphoreType.DMA,
        pltpu.SemaphoreType.DMA,
        pltpu.SemaphoreType.DMA,
        pltpu.SemaphoreType.DMA,
        pltpu.SemaphoreType.DMA,
        pltpu.SemaphoreType.DMA,
        pltpu.SemaphoreType.DMA,
    ],
  )


def _kB_body(y_hbm, ridx_hbm, scale_hbm, dst_hbm, acc_hbm,
             acc_sh, rows, ridxs, scales, dstv2,
             semI0, semI1, semI2, semG0, semG1, semG2, semS0, semS1):
    cid = lax.axis_index("c")
    sid = lax.axis_index("s")
    wid = _wid()
    semI = (semI0, semI1, semI2)
    semG = (semG0, semG1, semG2)
    semS = (semS0, semS1)

    @pl.when(sid < WB_TILES)
    def _():
        def _zfill(j, _):
            for k in range(D // 16):
                rows[j, pl.ds(k * 16, 16)] = jnp.zeros((16,), jnp.float32)
            return 0

        lax.fori_loop(0, ZROWS, _zfill, 0)
        for c in range(WB_ROWS // ZROWS):
            pltpu.sync_copy(
                rows.at[pl.ds(0, ZROWS)],
                acc_sh.at[pl.ds(sid * WB_ROWS + c * ZROWS, ZROWS)])

    plsc.subcore_barrier()

    base = wid * EPT

    def _off(x):
        return x if isinstance(x, int) else pl.multiple_of(x, 8)

    def _sel(v, K, fn):
        if isinstance(v, int):
            fn(v % K)
        else:
            m = v % K
            for i in range(K):
                pl.when(m == i)(functools.partial(fn, i))

    def fire_stage(s):
        b = _off(base + s * EB)
        po = _off((s % RSTG) * EB)

        def _f(i):
            pltpu.async_copy(ridx_hbm.at[pl.ds(b, EB)],
                             ridxs.at[pl.ds(po, EB)], semI[i])
            pltpu.async_copy(scale_hbm.at[pl.ds(b, EB)],
                             scales.at[pl.ds(po, EB)], semI[i])
            pltpu.async_copy(dst_hbm.at[pl.ds(b, EB)],
                             dstv2.at[s % RSTG], semI[i])
        _sel(s, 3, _f)

    def wait_stage(s):
        po = _off((s % RSTG) * EB)

        def _f(i):
            pltpu.make_async_copy(ridx_hbm.at[pl.ds(0, EB)],
                                  ridxs.at[pl.ds(po, EB)], semI[i]).wait()
            pltpu.make_async_copy(scale_hbm.at[pl.ds(0, EB)],
                                  scales.at[pl.ds(po, EB)], semI[i]).wait()
            pltpu.make_async_copy(dst_hbm.at[pl.ds(0, EB)],
                                  dstv2.at[s % RSTG], semI[i]).wait()
        _sel(s, 3, _f)

    def fire_gather(s):
        po = _off((s % RSTG) * EB)
        ro = _off((s % RROWS) * EB)

        def _f(i):
            pltpu.async_copy(y_hbm.at[ridxs.at[pl.ds(po, EB)]],
                             rows.at[pl.ds(ro, EB)], semG[i])
        _sel(s, 3, _f)

    def wait_gather(s):
        po = _off((s % RSTG) * EB)
        ro = _off((s % RROWS) * EB)

        def _f(i):
            pltpu.make_async_copy(y_hbm.at[ridxs.at[pl.ds(po, EB)]],
                                  rows.at[pl.ds(ro, EB)], semG[i]).wait()
        _sel(s, 3, _f)

    def fire_scatter(s):
        ro = _off((s % RROWS) * EB)

        def _f(i):
            pltpu.async_copy(rows.at[pl.ds(ro, EB)],
                             acc_sh.at[dstv2.at[s % RSTG]], semS[i], add=True)
        _sel(s, 2, _f)

    def wait_scatter(s):
        ro = _off((s % RROWS) * EB)

        def _f(i):
            pltpu.make_async_copy(rows.at[pl.ds(ro, EB)],
                                  acc_sh.at[dstv2.at[s % RSTG]],
                                  semS[i]).wait()
        _sel(s, 2, _f)

    def scale_block(s):
        off = _off((s % RROWS) * EB)
        soff = _off((s % RSTG) * EB)

        def body(jj, _):
            s16 = scales[pl.ds(soff + jj * 16, 16)]
            for i in range(16):
                j = off + jj * 16 + i
                sv = s16[i]
                for k in range(D // 16):
                    sl = pl.ds(k * 16, 16)
                    rows[j, sl] = rows[j, sl] * sv
            return 0

        lax.fori_loop(0, EB // 16, body, 0)

    fire_stage(0)
    fire_stage(1)

    def _iter(s, _):
        wait_stage(s)

        @pl.when(s >= 3)
        def _():
            wait_scatter(s - 3)

        fire_gather(s)

        @pl.when(s >= 2)
        def _():
            wait_gather(s - 2)
            scale_block(s - 2)
            fire_scatter(s - 2)

        @pl.when(s < NBLK_E - 2)
        def _():
            fire_stage(s + 2)

        return 0

    lax.fori_loop(0, NBLK_E, _iter, 0)
    for s in (NBLK_E - 2, NBLK_E - 1):
        wait_gather(s)
        scale_block(s)
        fire_scatter(s)
    for s in (NBLK_E - 3, NBLK_E - 2, NBLK_E - 1):
        wait_scatter(s)

    plsc.subcore_barrier()

    @pl.when(sid < WB_TILES)
    def _():
        pltpu.sync_copy(acc_sh.at[pl.ds(sid * WB_ROWS, WB_ROWS)],
                        acc_hbm.at[cid, pl.ds(sid * WB_ROWS, WB_ROWS)])


NB = 1000
NBLK = N // NB


def _kA_body(x_ref, w_ref, root_ref, b_ref, y_ref, oroot_ref):
    r = pl.program_id(1)
    xb = x_ref[...]

    @pl.when(r == 0)
    def _():
        oroot_ref[...] = (
            jnp.dot(xb, root_ref[...], preferred_element_type=jnp.float32)
            + b_ref[...]
        )

    y_ref[...] = jnp.dot(xb, w_ref[0], preferred_element_type=jnp.float32)


def _kA(x, W, root, b):
    return pl.pallas_call(
        _kA_body,
        grid=(NBLK, R),
        in_specs=[
            pl.BlockSpec((NB, D), lambda i, r: (i, 0)),
            pl.BlockSpec((1, D, D), lambda i, r: (r, 0, 0)),
            pl.BlockSpec((D, D), lambda i, r: (0, 0)),
            pl.BlockSpec((1, D), lambda i, r: (0, 0)),
        ],
        out_specs=[
            pl.BlockSpec((NB, D), lambda i, r: (r * NBLK + i, 0)),
            pl.BlockSpec((NB, D), lambda i, r: (i, 0)),
        ],
        out_shape=[
            jax.ShapeDtypeStruct((R * N, D), jnp.float32),
            jax.ShapeDtypeStruct((N, D), jnp.float32),
        ],
    )(x, W, root, b)


def _kC_body(rin_ref, acc_ref, w_ref, root_ref, b_ref, y_ref, oroot_ref, h_scr):
    r = pl.program_id(1)

    @pl.when(r == 0)
    def _():
        h = jax.nn.relu(rin_ref[...] + acc_ref[0] + acc_ref[1])
        h_scr[...] = h
        oroot_ref[...] = (
            jnp.dot(h, root_ref[...], preferred_element_type=jnp.float32)
            + b_ref[...]
        )

    y_ref[...] = jnp.dot(h_scr[...], w_ref[0], preferred_element_type=jnp.float32)


def _kC(rin, acc, W, root, b):
    return pl.pallas_call(
        _kC_body,
        grid=(NBLK, R),
        in_specs=[
            pl.BlockSpec((NB, D), lambda i, r: (i, 0)),
            pl.BlockSpec((NC, NB, D), lambda i, r: (0, i, 0)),
            pl.BlockSpec((1, D, D), lambda i, r: (r, 0, 0)),
            pl.BlockSpec((D, D), lambda i, r: (0, 0)),
            pl.BlockSpec((1, D), lambda i, r: (0, 0)),
        ],
        out_specs=[
            pl.BlockSpec((NB, D), lambda i, r: (r * NBLK + i, 0)),
            pl.BlockSpec((NB, D), lambda i, r: (i, 0)),
        ],
        out_shape=[
            jax.ShapeDtypeStruct((R * N, D), jnp.float32),
            jax.ShapeDtypeStruct((N, D), jnp.float32),
        ],
        scratch_shapes=[pltpu.VMEM((NB, D), jnp.float32)],
    )(rin, acc, W, root, b)


def _kD_body(rin_ref, acc_ref, batch_ref, out_ref, pool_scr, cnt_scr):
    i = pl.program_id(0)

    @pl.when(i == 0)
    def _():
        pool_scr[...] = jnp.zeros((G, D), jnp.float32)
        cnt_scr[...] = jnp.zeros((G, D), jnp.float32)

    h2 = rin_ref[...] + acc_ref[0] + acc_ref[1]
    ids = batch_ref[0, 0].reshape(NB, 1)
    one = (ids == lax.broadcasted_iota(jnp.int32, (NB, G), 1)).astype(jnp.float32)
    dn = (((0,), (0,)), ((), ()))
    pool_scr[...] += lax.dot_general(one, h2, dn,
                                     preferred_element_type=jnp.float32)
    cnt_scr[...] += lax.dot_general(one, jnp.ones((NB, D), jnp.float32), dn,
                                    preferred_element_type=jnp.float32)

    @pl.when(i == NBLK - 1)
    def _():
        out_ref[...] = pool_scr[...] / jnp.maximum(cnt_scr[...], 1.0)


def _kD(rin, acc, batch3):
    return pl.pallas_call(
        _kD_body,
        grid=(NBLK,),
        in_specs=[
            pl.BlockSpec((NB, D), lambda i: (i, 0)),
            pl.BlockSpec((NC, NB, D), lambda i: (0, i, 0)),
            pl.BlockSpec((1, 1, NB), lambda i: (i, 0, 0)),
        ],
        out_specs=pl.BlockSpec((G, D), lambda i: (0, 0)),
        out_shape=jax.ShapeDtypeStruct((G, D), jnp.float32),
        scratch_shapes=[
            pltpu.VMEM((G, D), jnp.float32),
            pltpu.VMEM((G, D), jnp.float32),
        ],
    )(rin, acc, batch3)


def kernel(x, edge_index, edge_type, batch, W1, root1, b1, W2, root2, b2):
    src = edge_index[0].astype(jnp.int32)
    dst = edge_index[1].astype(jnp.int32)
    et = edge_type.astype(jnp.int32)
    bt = batch.astype(jnp.int32).reshape(NBLK, 1, NB)

    scale, ridx = _k0()(src, et, dst)
    y1, oroot1 = _kA(x, W1, root1, b1.reshape(1, D))
    acc1 = _kB()(y1, ridx, scale, dst)
    y2, oroot2 = _kC(oroot1, acc1, W2, root2, b2.reshape(1, D))
    acc2 = _kB()(y2, ridx, scale, dst)
    return _kD(oroot2, acc2, bt)


# speedup vs baseline: 1.0131x; 1.0131x over previous
import functools

import jax
import jax.numpy as jnp
from jax import lax
from jax.experimental import pallas as pl
from jax.experimental.pallas import tpu as pltpu
from jax.experimental.pallas import tpu_sc as plsc

N = 10000
E = 320000
D = 128
R = 8
G = 64

NC = 2
NS = 16
NW = NC * NS

EB = 80
EPT_ALL = E // NS
EPT = E // NW
CNT_BINS = N * R
CNT_PAD = 81920
ZROWS = 40
WB_TILES = 10
WB_ROWS = N // WB_TILES

@functools.cache
def _mesh():
    return plsc.VectorSubcoreMesh(
        core_axis_name="c", subcore_axis_name="s", num_cores=NC, num_subcores=NS
    )


def _wid():
    return lax.axis_index("s") * NC + lax.axis_index("c")


EBA = 160
NBLK_A = EPT_ALL // EBA
NBLK_B = EPT // EB


@functools.cache
def _k0():
  return pl.kernel(
    _k0_body,
    out_type=[
        jax.ShapeDtypeStruct((E,), jnp.float32),
        jax.ShapeDtypeStruct((E,), jnp.int32),
    ],
    mesh=_mesh(),
    scratch_types=[
        pltpu.VMEM_SHARED((CNT_PAD,), jnp.float32),
        pltpu.VMEM((3 * EB,), jnp.float32),
        pltpu.VMEM((5120,), jnp.float32),
        pltpu.VMEM((2 * EBA,), jnp.int32),
        pltpu.VMEM((2 * EBA,), jnp.int32),
        pltpu.VMEM((2 * EB,), jnp.int32),
        pltpu.VMEM((4, EB), jnp.int32),
        pltpu.VMEM((3 * EB,), jnp.int32),
        pltpu.VMEM((EB,), jnp.float32),
        pltpu.VMEM((3 * EB,), jnp.float32),
        pltpu.VMEM((3 * EB,), jnp.int32),
        pltpu.SemaphoreType.DMA,
        pltpu.SemaphoreType.DMA,
        pltpu.SemaphoreType.DMA,
        pltpu.SemaphoreType.DMA,
        pltpu.SemaphoreType.DMA,
        pltpu.SemaphoreType.DMA,
        pltpu.SemaphoreType.DMA,
    ],
  )


def _k0_body(src_hbm, t_hbm, dst_hbm, scale_hbm, ridx_hbm,
             cnt_sh, cntv, zbuf, dstv, tv, srcv, bidxA, bidxB, ones,
             sbuf, rbuf, semI0, semI1, semG0, semG1, semG2, semS0, semS1):
    sid = lax.axis_index("s")
    semG = (semG0, semG1, semG2)

    def _off(x):
        return x if isinstance(x, int) else pl.multiple_of(x, 8)

    def _sel(p, q, fn):
        if isinstance(p, int):
            if p == q:
                fn()
        else:
            pl.when(p == q)(fn)

    def _zfill(j, _):
        zbuf[pl.ds(j * 16, 16)] = jnp.zeros((16,), jnp.float32)
        return 0

    lax.fori_loop(0, 5120 // 16, _zfill, 0)
    pltpu.sync_copy(zbuf, cnt_sh.at[pl.ds(sid * 5120, 5120)])
    for j in range(EB // 16):
        ones[pl.ds(j * 16, 16)] = jnp.ones((16,), jnp.float32)
    plsc.subcore_barrier()

    base_a = sid * EPT_ALL

    def fsA(s):
        p, b, po = s % 2, _off(base_a + s * EBA), _off((s % 2) * EBA)
        for semIx, q in ((semI0, 0), (semI1, 1)):
            def _f(semIx=semIx):
                pltpu.async_copy(dst_hbm.at[pl.ds(b, EBA)],
                                 dstv.at[pl.ds(po, EBA)], semIx)
                pltpu.async_copy(t_hbm.at[pl.ds(b, EBA)],
                                 tv.at[pl.ds(po, EBA)], semIx)
            _sel(p, q, _f)

    def wsA(s):
        p, po = s % 2, _off((s % 2) * EBA)
        for semIx, q in ((semI0, 0), (semI1, 1)):
            def _f(semIx=semIx):
                pltpu.make_async_copy(dst_hbm.at[pl.ds(0, EBA)],
                                      dstv.at[pl.ds(po, EBA)], semIx).wait()
                pltpu.make_async_copy(t_hbm.at[pl.ds(0, EBA)],
                                      tv.at[pl.ds(po, EBA)], semIx).wait()
            _sel(p, q, _f)

    def fscatA(s):
        p = s % 2
        for semSx, q in ((semS0, 0), (semS1, 1)):
            def _f(semSx=semSx):
                pltpu.async_copy(ones, cnt_sh.at[bidxA.at[2 * p]],
                                 semSx, add=True)
                pltpu.async_copy(ones, cnt_sh.at[bidxA.at[2 * p + 1]],
                                 semSx, add=True)
            _sel(p, q, _f)

    def wscatA(s):
        p = s % 2
        for semSx, q in ((semS0, 0), (semS1, 1)):
            def _f(semSx=semSx):
                pltpu.make_async_copy(ones, cnt_sh.at[bidxA.at[2 * p]],
                                      semSx).wait()
                pltpu.make_async_copy(ones, cnt_sh.at[bidxA.at[2 * p + 1]],
                                      semSx).wait()
            _sel(p, q, _f)

    fsA(0)

    def _iterA(s, _):
        p = s % 2
        po = _off(p * EBA)
        wsA(s)

        @pl.when(s >= 2)
        def _():
            wscatA(s - 2)

        for j in range(EBA // 16):
            sl = pl.ds(po + j * 16, 16)
            bidxA[2 * p + j // 5, pl.ds((j % 5) * 16, 16)] = (
                dstv[sl] * R + tv[sl])
        fscatA(s)

        @pl.when(s < NBLK_A - 1)
        def _():
            fsA(s + 1)

        return 0

    lax.fori_loop(0, NBLK_A, _iterA, 0)
    wscatA(NBLK_A - 2)
    wscatA(NBLK_A - 1)
    plsc.subcore_barrier()

    base_c = _wid() * EPT

    def fsB(s):
        p, b, po = s % 2, _off(base_c + s * EB), _off((s % 2) * EB)
        for semI, q in ((semI0, 0), (semI1, 1)):
            def _f(semI=semI):
                pltpu.async_copy(src_hbm.at[pl.ds(b, EB)],
                                 srcv.at[pl.ds(po, EB)], semI)
                pltpu.async_copy(t_hbm.at[pl.ds(b, EB)],
                                 tv.at[pl.ds(po, EB)], semI)
                pltpu.async_copy(dst_hbm.at[pl.ds(b, EB)],
                                 dstv.at[pl.ds(po, EB)], semI)
            _sel(p, q, _f)

    def wsB(s):
        p, po = s % 2, _off((s % 2) * EB)
        for semI, q in ((semI0, 0), (semI1, 1)):
            def _f(semI=semI):
                pltpu.make_async_copy(src_hbm.at[pl.ds(0, EB)],
                                      srcv.at[pl.ds(po, EB)], semI).wait()
                pltpu.make_async_copy(t_hbm.at[pl.ds(0, EB)],
                                      tv.at[pl.ds(po, EB)], semI).wait()
                pltpu.make_async_copy(dst_hbm.at[pl.ds(0, EB)],
                                      dstv.at[pl.ds(po, EB)], semI).wait()
            _sel(p, q, _f)

    def fgB(s):
        p, po = s % 2, _off((s % 2) * EB)
        for semGx, q in ((semG0, 0), (semG1, 1)):
            def _f(semGx=semGx):
                pltpu.async_copy(cnt_sh.at[bidxB.at[pl.ds(po, EB)]],
                                 cntv.at[pl.ds(po, EB)], semGx)
            _sel(p, q, _f)

    def wgB(s):
        p, po = s % 2, _off((s % 2) * EB)
        for semGx, q in ((semG0, 0), (semG1, 1)):
            def _f(semGx=semGx):
                pltpu.make_async_copy(cnt_sh.at[bidxB.at[pl.ds(po, EB)]],
                                      cntv.at[pl.ds(po, EB)], semGx).wait()
            _sel(p, q, _f)

    def fwB(s):
        p, b, po = s % 2, _off(base_c + s * EB), _off((s % 2) * EB)
        for semSx, q in ((semS0, 0), (semS1, 1)):
            def _f(semSx=semSx):
                pltpu.async_copy(sbuf.at[pl.ds(po, EB)],
                                 scale_hbm.at[pl.ds(b, EB)], semSx)
                pltpu.async_copy(rbuf.at[pl.ds(po, EB)],
                                 ridx_hbm.at[pl.ds(b, EB)], semSx)
            _sel(p, q, _f)

    def wwB(s):
        p, po = s % 2, _off((s % 2) * EB)
        for semSx, q in ((semS0, 0), (semS1, 1)):
            def _f(semSx=semSx):
                pltpu.make_async_copy(sbuf.at[pl.ds(po, EB)],
                                      scale_hbm.at[pl.ds(0, EB)],
                                      semSx).wait()
                pltpu.make_async_copy(rbuf.at[pl.ds(po, EB)],
                                      ridx_hbm.at[pl.ds(0, EB)],
                                      semSx).wait()
            _sel(p, q, _f)

    def emitB(s):
        po = _off((s % 2) * EB)
        for j in range(EB // 16):
            sl = pl.ds(po + j * 16, 16)
            sbuf[sl] = 1.0 / jnp.maximum(cntv[sl], 1.0)

    fsB(0)

    def _iterB(s, _):
        p = s % 2
        po = _off(p * EB)
        wsB(s)

        @pl.when(s >= 2)
        def _():
            wwB(s - 2)

        for j in range(EB // 16):
            sl = pl.ds(po + j * 16, 16)
            t16 = tv[sl]
            rbuf[sl] = t16 * N + srcv[sl]
            bidxB[sl] = dstv[sl] * R + t16
        fgB(s)

        @pl.when(s >= 1)
        def _():
            wgB(s - 1)
            emitB(s - 1)
            fwB(s - 1)

        @pl.when(s < NBLK_B - 1)
        def _():
            fsB(s + 1)

        return 0

    lax.fori_loop(0, NBLK_B, _iterB, 0)
    s_last = NBLK_B - 1
    wgB(s_last)
    emitB(s_last)
    fwB(s_last)
    wwB(s_last - 1)
    wwB(s_last)


NBLK_E = EPT // EB


RROWS = 4
RSTG = 5


@functools.cache
def _kB():
  return pl.kernel(
    _kB_body,
    out_type=jax.ShapeDtypeStruct((NC, N, D), jnp.float32),
    mesh=_mesh(),
    scratch_types=[
        pltpu.VMEM_SHARED((N, D), jnp.float32),
        pltpu.VMEM((RROWS * EB, D), jnp.float32),
        pltpu.VMEM((RSTG * EB,), jnp.int32),
        pltpu.VMEM((RSTG * EB,), jnp.float32),
        pltpu.VMEM((RSTG, EB), jnp.int32),
        pltpu.SemaphoreType.DMA,
        pltpu.SemaphoreType.DMA,
        pltpu.SemaphoreType.DMA,
        pltpu.SemaphoreType.DMA,
        pltpu.SemaphoreType.DMA,
        pltpu.SemaphoreType.DMA,
        pltpu.SemaphoreType.DMA,
        pltpu.SemaphoreType.DMA,
    ],
  )


def _kB_body(y_hbm, ridx_hbm, scale_hbm, dst_hbm, acc_hbm,
             acc_sh, rows, ridxs, scales, dstv2,
             semI0, semI1, semI2, semG0, semG1, semG2, semS0, semS1):
    cid = lax.axis_index("c")
    sid = lax.axis_index("s")
    wid = _wid()
    semI = (semI0, semI1, semI2)
    semG = (semG0, semG1, semG2)
    semS = (semS0, semS1)

    @pl.when(sid < WB_TILES)
    def _():
        def _zfill(j, _):
            for k in range(D // 16):
                rows[j, pl.ds(k * 16, 16)] = jnp.zeros((16,), jnp.float32)
            return 0

        lax.fori_loop(0, ZROWS, _zfill, 0)
        for c in range(WB_ROWS // ZROWS):
            pltpu.sync_copy(
                rows.at[pl.ds(0, ZROWS)],
                acc_sh.at[pl.ds(sid * WB_ROWS + c * ZROWS, ZROWS)])

    plsc.subcore_barrier()

    base = wid * EPT

    def _off(x):
        return x if isinstance(x, int) else pl.multiple_of(x, 8)

    def _sel(v, K, fn):
        if isinstance(v, int):
            fn(v % K)
        else:
            m = v % K
            for i in range(K):
                pl.when(m == i)(functools.partial(fn, i))

    def fire_stage(s):
        b = _off(base + s * EB)
        po = _off((s % RSTG) * EB)

        def _f(i):
            pltpu.async_copy(ridx_hbm.at[pl.ds(b, EB)],
                             ridxs.at[pl.ds(po, EB)], semI[i])
            pltpu.async_copy(scale_hbm.at[pl.ds(b, EB)],
                             scales.at[pl.ds(po, EB)], semI[i])
            pltpu.async_copy(dst_hbm.at[pl.ds(b, EB)],
                             dstv2.at[s % RSTG], semI[i])
        _sel(s, 3, _f)

    def wait_stage(s):
        po = _off((s % RSTG) * EB)

        def _f(i):
            pltpu.make_async_copy(ridx_hbm.at[pl.ds(0, EB)],
                                  ridxs.at[pl.ds(po, EB)], semI[i]).wait()
            pltpu.make_async_copy(scale_hbm.at[pl.ds(0, EB)],
                                  scales.at[pl.ds(po, EB)], semI[i]).wait()
            pltpu.make_async_copy(dst_hbm.at[pl.ds(0, EB)],
                                  dstv2.at[s % RSTG], semI[i]).wait()
        _sel(s, 3, _f)

    def fire_gather(s):
        po = _off((s % RSTG) * EB)
        ro = _off((s % RROWS) * EB)

        def _f(i):
            pltpu.async_copy(y_hbm.at[ridxs.at[pl.ds(po, EB)]],
                             rows.at[pl.ds(ro, EB)], semG[i])
        _sel(s, 3, _f)

    def wait_gather(s):
        po = _off((s % RSTG) * EB)
        ro = _off((s % RROWS) * EB)

        def _f(i):
            pltpu.make_async_copy(y_hbm.at[ridxs.at[pl.ds(po, EB)]],
                                  rows.at[pl.ds(ro, EB)], semG[i]).wait()
        _sel(s, 3, _f)

    def fire_scatter(s):
        ro = _off((s % RROWS) * EB)

        def _f(i):
            pltpu.async_copy(rows.at[pl.ds(ro, EB)],
                             acc_sh.at[dstv2.at[s % RSTG]], semS[i], add=True)
        _sel(s, 2, _f)

    def wait_scatter(s):
        ro = _off((s % RROWS) * EB)

        def _f(i):
            pltpu.make_async_copy(rows.at[pl.ds(ro, EB)],
                                  acc_sh.at[dstv2.at[s % RSTG]],
                                  semS[i]).wait()
        _sel(s, 2, _f)

    def scale_block(s):
        off = _off((s % RROWS) * EB)
        soff = _off((s % RSTG) * EB)

        def body(jj, _):
            s16 = scales[pl.ds(soff + jj * 16, 16)]
            for i in range(16):
                j = off + jj * 16 + i
                sv = s16[i]
                for k in range(D // 16):
                    sl = pl.ds(k * 16, 16)
                    rows[j, sl] = rows[j, sl] * sv
            return 0

        lax.fori_loop(0, EB // 16, body, 0)

    fire_stage(0)
    fire_stage(1)

    def _iter(s, _):
        wait_stage(s)

        @pl.when(s >= 3)
        def _():
            wait_scatter(s - 3)

        fire_gather(s)

        @pl.when(s >= 2)
        def _():
            wait_gather(s - 2)
            scale_block(s - 2)
            fire_scatter(s - 2)

        @pl.when(s < NBLK_E - 2)
        def _():
            fire_stage(s + 2)

        return 0

    lax.fori_loop(0, NBLK_E, _iter, 0)
    for s in (NBLK_E - 2, NBLK_E - 1):
        wait_gather(s)
        scale_block(s)
        fire_scatter(s)
    for s in (NBLK_E - 3, NBLK_E - 2, NBLK_E - 1):
        wait_scatter(s)

    plsc.subcore_barrier()

    @pl.when(sid < WB_TILES)
    def _():
        pltpu.sync_copy(acc_sh.at[pl.ds(sid * WB_ROWS, WB_ROWS)],
                        acc_hbm.at[cid, pl.ds(sid * WB_ROWS, WB_ROWS)])


NB = 1000
NBLK = N // NB


def _kA_body(x_ref, w_ref, root_ref, b_ref, y_ref, oroot_ref):
    r = pl.program_id(1)
    xb = x_ref[...]

    @pl.when(r == 0)
    def _():
        oroot_ref[...] = (
            jnp.dot(xb, root_ref[...], preferred_element_type=jnp.float32)
            + b_ref[...]
        )

    y_ref[...] = jnp.dot(xb, w_ref[0], preferred_element_type=jnp.float32)


def _kA(x, W, root, b):
    return pl.pallas_call(
        _kA_body,
        grid=(NBLK, R),
        in_specs=[
            pl.BlockSpec((NB, D), lambda i, r: (i, 0)),
            pl.BlockSpec((1, D, D), lambda i, r: (r, 0, 0)),
            pl.BlockSpec((D, D), lambda i, r: (0, 0)),
            pl.BlockSpec((1, D), lambda i, r: (0, 0)),
        ],
        out_specs=[
            pl.BlockSpec((NB, D), lambda i, r: (r * NBLK + i, 0)),
            pl.BlockSpec((NB, D), lambda i, r: (i, 0)),
        ],
        out_shape=[
            jax.ShapeDtypeStruct((R * N, D), jnp.float32),
            jax.ShapeDtypeStruct((N, D), jnp.float32),
        ],
    )(x, W, root, b)


def _kC_body(rin_ref, acc_ref, w_ref, root_ref, b_ref, y_ref, oroot_ref, h_scr):
    r = pl.program_id(1)

    @pl.when(r == 0)
    def _():
        h = jax.nn.relu(rin_ref[...] + acc_ref[0] + acc_ref[1])
        h_scr[...] = h
        oroot_ref[...] = (
            jnp.dot(h, root_ref[...], preferred_element_type=jnp.float32)
            + b_ref[...]
        )

    y_ref[...] = jnp.dot(h_scr[...], w_ref[0], preferred_element_type=jnp.float32)


def _kC(rin, acc, W, root, b):
    return pl.pallas_call(
        _kC_body,
        grid=(NBLK, R),
        in_specs=[
            pl.BlockSpec((NB, D), lambda i, r: (i, 0)),
            pl.BlockSpec((NC, NB, D), lambda i, r: (0, i, 0)),
            pl.BlockSpec((1, D, D), lambda i, r: (r, 0, 0)),
            pl.BlockSpec((D, D), lambda i, r: (0, 0)),
            pl.BlockSpec((1, D), lambda i, r: (0, 0)),
        ],
        out_specs=[
            pl.BlockSpec((NB, D), lambda i, r: (r * NBLK + i, 0)),
            pl.BlockSpec((NB, D), lambda i, r: (i, 0)),
        ],
        out_shape=[
            jax.ShapeDtypeStruct((R * N, D), jnp.float32),
            jax.ShapeDtypeStruct((N, D), jnp.float32),
        ],
        scratch_shapes=[pltpu.VMEM((NB, D), jnp.float32)],
    )(rin, acc, W, root, b)


def _kD_body(rin_ref, acc_ref, batch_ref, out_ref, pool_scr, cnt_scr):
    i = pl.program_id(0)

    @pl.when(i == 0)
    def _():
        pool_scr[...] = jnp.zeros((G, D), jnp.float32)
        cnt_scr[...] = jnp.zeros((G, D), jnp.float32)

    h2 = rin_ref[...] + acc_ref[0] + acc_ref[1]
    ids = batch_ref[0, 0].reshape(NB, 1)
    one = (ids == lax.broadcasted_iota(jnp.int32, (NB, G), 1)).astype(jnp.float32)
    dn = (((0,), (0,)), ((), ()))
    pool_scr[...] += lax.dot_general(one, h2, dn,
                                     preferred_element_type=jnp.float32)
    cnt_scr[...] += lax.dot_general(one, jnp.ones((NB, D), jnp.float32), dn,
                                    preferred_element_type=jnp.float32)

    @pl.when(i == NBLK - 1)
    def _():
        out_ref[...] = pool_scr[...] / jnp.maximum(cnt_scr[...], 1.0)


def _kD(rin, acc, batch3):
    return pl.pallas_call(
        _kD_body,
        grid=(NBLK,),
        in_specs=[
            pl.BlockSpec((NB, D), lambda i: (i, 0)),
            pl.BlockSpec((NC, NB, D), lambda i: (0, i, 0)),
            pl.BlockSpec((1, 1, NB), lambda i: (i, 0, 0)),
        ],
        out_specs=pl.BlockSpec((G, D), lambda i: (0, 0)),
        out_shape=jax.ShapeDtypeStruct((G, D), jnp.float32),
        scratch_shapes=[
            pltpu.VMEM((G, D), jnp.float32),
            pltpu.VMEM((G, D), jnp.float32),
        ],
    )(rin, acc, batch3)


def kernel(x, edge_index, edge_type, batch, W1, root1, b1, W2, root2, b2):
    src = edge_index[0].astype(jnp.int32)
    dst = edge_index[1].astype(jnp.int32)
    et = edge_type.astype(jnp.int32)
    bt = batch.astype(jnp.int32).reshape(NBLK, 1, NB)

    scale, ridx = _k0()(src, et, dst)
    y1, oroot1 = _kA(x, W1, root1, b1.reshape(1, D))
    acc1 = _kB()(y1, ridx, scale, dst)
    y2, oroot2 = _kC(oroot1, acc1, W2, root2, b2.reshape(1, D))
    acc2 = _kB()(y2, ridx, scale, dst)
    return _kD(oroot2, acc2, bt)
